# Initial kernel scaffold; baseline (speedup 1.0000x reference)
import jax
import jax.numpy as jnp
from jax.experimental import pallas as pl


def kernel(inputs, table):
    raise NotImplementedError("write your pallas kernel here")


# speedup vs baseline: 2.4069x; 2.4069x over previous
import functools

import numpy as np
import jax
import jax.numpy as jnp
from jax import lax
from jax.experimental import pallas as pl
from jax.experimental.pallas import tpu as pltpu
from jax.experimental.pallas import tpu_sc as plsc

HIDDEN = 64
SEQ_LEN = 200
BATCH = 4096
N_ROWS = BATCH * SEQ_LEN

NC = 2
NS = 16
NW = NC * NS
RPW = N_ROWS // NW

CHUNK = 256
G = 128
NCHUNK = RPW // CHUNK
PE_EXT_ROWS = SEQ_LEN + CHUNK


def _pe_extended() -> np.ndarray:
    seq_pos = np.arange(SEQ_LEN, dtype=np.float32) + 1.0
    power = np.arange(0, HIDDEN, 2, dtype=np.float32) / HIDDEN
    divisor = 10000.0 ** power
    ang = seq_pos[:, None] / divisor[None, :]
    pe = np.stack((np.sin(ang), np.cos(ang)), axis=-1)
    pe = pe.reshape(SEQ_LEN, HIDDEN)
    rows = np.arange(PE_EXT_ROWS) % SEQ_LEN
    return np.ascontiguousarray(pe[rows])


_PE_EXT = _pe_extended()


def _sc_body(idx_hbm, table_hbm, pe_hbm, out_hbm, idx_v, rows_v, pe_v, gsem):
    wid = lax.axis_index("s") * NC + lax.axis_index("c")
    base = wid * RPW
    pltpu.sync_copy(pe_hbm, pe_v)

    @pl.loop(0, NCHUNK)
    def _chunk(c):
        cbase = base + c * CHUNK
        pltpu.sync_copy(idx_hbm.at[pl.ds(cbase, CHUNK)], idx_v)
        cp0 = pltpu.async_copy(
            table_hbm.at[idx_v.at[pl.ds(0, G)]], rows_v.at[pl.ds(0, G)], gsem)
        cp1 = pltpu.async_copy(
            table_hbm.at[idx_v.at[pl.ds(G, G)]], rows_v.at[pl.ds(G, G)], gsem)
        cp0.wait()
        cp1.wait()
        p = lax.rem(c * CHUNK, SEQ_LEN)

        @pl.loop(0, CHUNK)
        def _row(r):
            for h in range(HIDDEN // 16):
                plsc.addupdate(
                    rows_v.at[r, pl.ds(h * 16, 16)],
                    pe_v[p + r, pl.ds(h * 16, 16)],
                )

        pltpu.sync_copy(rows_v, out_hbm.at[pl.ds(cbase, CHUNK)])


@jax.jit
def _sc_embed(idx, table, pe_ext):
    mesh = plsc.VectorSubcoreMesh(
        core_axis_name="c", subcore_axis_name="s", num_cores=NC, num_subcores=NS)
    fn = functools.partial(
        pl.kernel,
        out_type=jax.ShapeDtypeStruct((N_ROWS, HIDDEN), jnp.float32),
        mesh=mesh,
        scratch_types=[
            pltpu.VMEM((CHUNK,), jnp.int32),
            pltpu.VMEM((CHUNK, HIDDEN), jnp.float32),
            pltpu.VMEM((PE_EXT_ROWS, HIDDEN), jnp.float32),
            pltpu.SemaphoreType.DMA,
        ],
        compiler_params=pltpu.CompilerParams(use_tc_tiling_on_sc=False),
    )(_sc_body)
    return fn(idx, table, pe_ext)


def kernel(inputs, table):
    idx = inputs.reshape(-1).astype(jnp.int32)
    table = table.astype(jnp.float32)
    pe_ext = jnp.asarray(_PE_EXT, dtype=jnp.float32)
    out = _sc_embed(idx, table, pe_ext)
    return out.reshape(BATCH, SEQ_LEN, HIDDEN)


# speedup vs baseline: 2.8837x; 1.1981x over previous
import functools

import numpy as np
import jax
import jax.numpy as jnp
from jax import lax
from jax.experimental import pallas as pl
from jax.experimental.pallas import tpu as pltpu
from jax.experimental.pallas import tpu_sc as plsc

HIDDEN = 64
SEQ_LEN = 200
BATCH = 4096
N_ROWS = BATCH * SEQ_LEN

NC = 2
NS = 16
NW = NC * NS
RPW = N_ROWS // NW

CHUNK = 256
G = 128
NCHUNK = RPW // CHUNK
PE_EXT_ROWS = SEQ_LEN + CHUNK


def _pe_extended() -> np.ndarray:
    seq_pos = np.arange(SEQ_LEN, dtype=np.float32) + 1.0
    power = np.arange(0, HIDDEN, 2, dtype=np.float32) / HIDDEN
    divisor = 10000.0 ** power
    ang = seq_pos[:, None] / divisor[None, :]
    pe = np.stack((np.sin(ang), np.cos(ang)), axis=-1)
    pe = pe.reshape(SEQ_LEN, HIDDEN)
    rows = np.arange(PE_EXT_ROWS) % SEQ_LEN
    return np.ascontiguousarray(pe[rows])


_PE_EXT = _pe_extended()


NG = CHUNK // G


def _sc_body(idx_hbm, table_hbm, pe_hbm, out_hbm,
             idx_v, rows_v, pe_v, gsem0, gsem1, osem0, osem1):
    wid = lax.axis_index("s") * NC + lax.axis_index("c")
    base = wid * RPW
    gsem = (gsem0, gsem1)
    osem = (osem0, osem1)
    pltpu.sync_copy(idx_hbm.at[pl.ds(base, RPW)], idx_v)
    pltpu.sync_copy(pe_hbm, pe_v)

    def gather_descs(c, b):
        return [
            pltpu.make_async_copy(
                table_hbm.at[idx_v.at[pl.ds(c * CHUNK + g * G, G)]],
                rows_v.at[b, pl.ds(g * G, G)],
                gsem[b],
            )
            for g in range(NG)
        ]

    def out_desc(c, b):
        return pltpu.make_async_copy(
            rows_v.at[b], out_hbm.at[pl.ds(base + c * CHUNK, CHUNK)], osem[b])

    for d in gather_descs(0, 0):
        d.start()

    @pl.loop(0, NCHUNK, step=2)
    def _chunk2(c0):
        for b in range(2):
            c = c0 + b
            @pl.when(c >= 1)
            def _():
                out_desc(c - 1, 1 - b).wait()

            @pl.when(c + 1 < NCHUNK)
            def _():
                for d in gather_descs(c + 1, 1 - b):
                    d.start()

            for d in gather_descs(c, b):
                d.wait()
            p = lax.rem(c * CHUNK, SEQ_LEN)

            @pl.loop(0, CHUNK, unroll=4)
            def _row(r):
                for h in range(HIDDEN // 16):
                    plsc.addupdate(
                        rows_v.at[b, r, pl.ds(h * 16, 16)],
                        pe_v[p + r, pl.ds(h * 16, 16)],
                    )

            out_desc(c, b).start()

    out_desc(NCHUNK - 1, (NCHUNK - 1) % 2).wait()


@jax.jit
def _sc_embed(idx, table, pe_ext):
    mesh = plsc.VectorSubcoreMesh(
        core_axis_name="c", subcore_axis_name="s", num_cores=NC, num_subcores=NS)
    fn = functools.partial(
        pl.kernel,
        out_type=jax.ShapeDtypeStruct((N_ROWS, HIDDEN), jnp.float32),
        mesh=mesh,
        scratch_types=[
            pltpu.VMEM((RPW,), jnp.int32),
            pltpu.VMEM((2, CHUNK, HIDDEN), jnp.float32),
            pltpu.VMEM((PE_EXT_ROWS, HIDDEN), jnp.float32),
            pltpu.SemaphoreType.DMA,
            pltpu.SemaphoreType.DMA,
            pltpu.SemaphoreType.DMA,
            pltpu.SemaphoreType.DMA,
        ],
        compiler_params=pltpu.CompilerParams(use_tc_tiling_on_sc=False),
    )(_sc_body)
    return fn(idx, table, pe_ext)


def kernel(inputs, table):
    idx = inputs.reshape(-1).astype(jnp.int32)
    table = table.astype(jnp.float32)
    pe_ext = jnp.asarray(_PE_EXT, dtype=jnp.float32)
    out = _sc_embed(idx, table, pe_ext)
    return out.reshape(BATCH, SEQ_LEN, HIDDEN)


# speedup vs baseline: 4.0029x; 1.3881x over previous
import functools

import numpy as np
import jax
import jax.numpy as jnp
from jax import lax
from jax.experimental import pallas as pl
from jax.experimental.pallas import tpu as pltpu
from jax.experimental.pallas import tpu_sc as plsc

HIDDEN = 64
SEQ_LEN = 200
BATCH = 4096

NC = 2
NS = 16
NW = NC * NS
SPW = BATCH // NW
RPW = SPW * SEQ_LEN

GS = (128, 72)
GOFF = (0, 128)


def _pe_table() -> np.ndarray:
    seq_pos = np.arange(SEQ_LEN, dtype=np.float32) + 1.0
    power = np.arange(0, HIDDEN, 2, dtype=np.float32) / HIDDEN
    divisor = 10000.0 ** power
    ang = seq_pos[:, None] / divisor[None, :]
    pe = np.stack((np.sin(ang), np.cos(ang)), axis=-1)
    return np.ascontiguousarray(pe.reshape(SEQ_LEN, HIDDEN))


_PE = _pe_table()


def _sc_body(idx_hbm, table_hbm, pe_hbm, out_hbm,
             idx_v, rows_v, pe_v, gsem0, gsem1, osem0, osem1):
    wid = lax.axis_index("s") * NC + lax.axis_index("c")
    base = wid * RPW
    gsem = (gsem0, gsem1)
    osem = (osem0, osem1)
    pltpu.sync_copy(idx_hbm.at[pl.ds(base, RPW)], idx_v)
    pltpu.sync_copy(pe_hbm, pe_v)

    def gather_descs(c, b):
        return [
            pltpu.make_async_copy(
                table_hbm.at[idx_v.at[pl.ds(c * SEQ_LEN + off, n)]],
                rows_v.at[b, pl.ds(off, n)],
                gsem[b],
            )
            for off, n in zip(GOFF, GS)
        ]

    def out_desc(c, b):
        return pltpu.make_async_copy(
            rows_v.at[b], out_hbm.at[wid * SPW + c], osem[b])

    for d in gather_descs(0, 0):
        d.start()

    @pl.loop(0, SPW, step=2)
    def _chunk2(c0):
        for b in range(2):
            c = c0 + b
            @pl.when(c >= 1)
            def _():
                out_desc(c - 1, 1 - b).wait()

            @pl.when(c + 1 < SPW)
            def _():
                for d in gather_descs(c + 1, 1 - b):
                    d.start()

            for d in gather_descs(c, b):
                d.wait()

            @pl.loop(0, SEQ_LEN, unroll=4)
            def _row(r):
                for h in range(HIDDEN // 16):
                    plsc.addupdate(
                        rows_v.at[b, r, pl.ds(h * 16, 16)],
                        pe_v[r, pl.ds(h * 16, 16)],
                    )

            out_desc(c, b).start()

    out_desc(SPW - 1, (SPW - 1) % 2).wait()


@jax.jit
def _sc_embed(idx, table, pe):
    mesh = plsc.VectorSubcoreMesh(
        core_axis_name="c", subcore_axis_name="s", num_cores=NC, num_subcores=NS)
    fn = functools.partial(
        pl.kernel,
        out_type=jax.ShapeDtypeStruct((BATCH, SEQ_LEN, HIDDEN), jnp.float32),
        mesh=mesh,
        scratch_types=[
            pltpu.VMEM((RPW,), jnp.int32),
            pltpu.VMEM((2, SEQ_LEN, HIDDEN), jnp.float32),
            pltpu.VMEM((SEQ_LEN, HIDDEN), jnp.float32),
            pltpu.SemaphoreType.DMA,
            pltpu.SemaphoreType.DMA,
            pltpu.SemaphoreType.DMA,
            pltpu.SemaphoreType.DMA,
        ],
        compiler_params=pltpu.CompilerParams(use_tc_tiling_on_sc=False),
    )(_sc_body)
    return fn(idx, table, pe)


def kernel(inputs, table):
    idx = inputs.reshape(-1).astype(jnp.int32)
    table = table.astype(jnp.float32)
    pe = jnp.asarray(_PE, dtype=jnp.float32)
    return _sc_embed(idx, table, pe)
